# Initial kernel scaffold; baseline (speedup 1.0000x reference)
import jax
import jax.numpy as jnp
from jax.experimental import pallas as pl


def kernel(x, edge_index, W, a, bias):
    raise NotImplementedError("write your pallas kernel here")


# speedup vs baseline: 175.7593x; 175.7593x over previous
import functools

import jax
import jax.numpy as jnp
from jax import lax
from jax.experimental import pallas as pl
from jax.experimental.pallas import tpu as pltpu
from jax.experimental.pallas import tpu_sc as plsc

N = 10000
E = 320000
H = 8
HD = 16
IN_F_ = 128
NC = 2
NS = 16
NW = NC * NS
EPT = E // NW
PW = 80
ROWS = EPT // PW
GPR = PW // 16
CP = 25
NCHUNK = ROWS // CP


def _prep_body(x_ref, wt_ref, a1_ref, a2_ref, xs_ref, q1_ref, q2_ref,
               mv_ref, ws_ref):
    x = x_ref[...]
    xs = jnp.sum(x, axis=1, keepdims=True)
    xs_ref[...] = xs
    wsf = jnp.sum(wt_ref[...], axis=0, keepdims=True)
    ws_ref[...] = wsf

    def rnd(z):
        return z.astype(jnp.bfloat16).astype(jnp.float32)

    p2 = rnd(xs * wsf)
    grp = lax.broadcasted_iota(jnp.int32, (H * HD, H), 0) // HD
    hh = lax.broadcasted_iota(jnp.int32, (H * HD, H), 1)
    g = jnp.where(grp == hh, 1.0, 0.0)
    hp = jax.lax.Precision.HIGHEST
    q1 = jnp.dot(p2 * rnd(a1_ref[...]), g,
                 preferred_element_type=jnp.float32, precision=hp)
    q2 = jnp.dot(p2 * rnd(a2_ref[...]), g,
                 preferred_element_type=jnp.float32, precision=hp)
    q1_ref[...] = q1
    q2_ref[...] = q2
    t = (jnp.max(q1, axis=0, keepdims=True)
         + jnp.max(q2, axis=0, keepdims=True))
    mv_ref[...] = jnp.maximum(t, 0.2 * t)


def _edge_body(xs_hbm, src_hbm, dst_hbm, mb_hbm, q1_hbm, q2_hbm, zero_hbm,
               sp_hbm, zt_hbm,
               xs_v, src_v, dst_v, q1g, q2g, stage_v, mb_v, zv, s_sh,
               sem_g, sem_s):
    cid = lax.axis_index("c")
    sid = lax.axis_index("s")
    wid = sid * NC + cid

    pltpu.sync_copy(xs_hbm, xs_v)
    pltpu.sync_copy(src_hbm.at[wid], src_v)
    pltpu.sync_copy(dst_hbm.at[wid], dst_v)
    pltpu.sync_copy(mb_hbm, mb_v)
    @pl.when(sid < 10)
    def _zero():
        pltpu.sync_copy(zero_hbm.at[pl.ds(sid * 1000, 1000)],
                        s_sh.at[pl.ds(sid * 1000, 1000)])
    plsc.subcore_barrier()

    lane = lax.broadcasted_iota(jnp.int32, (16,), 0)
    mh = [mb_v[h] for h in range(H)]
    zacc = (lane * 0.0,) * H

    for chunk in range(NCHUNK):
        base = chunk * CP

        def g_start(r, carry):
            pltpu.make_async_copy(q1_hbm.at[src_v.at[base + r]],
                                  q1g.at[r], sem_g).start()
            pltpu.make_async_copy(q2_hbm.at[dst_v.at[base + r]],
                                  q2g.at[r], sem_g).start()
            return carry

        def g_wait(r, carry):
            pltpu.make_async_copy(q1_hbm.at[src_v.at[base + r]],
                                  q1g.at[r], sem_g).wait()
            pltpu.make_async_copy(q2_hbm.at[dst_v.at[base + r]],
                                  q2g.at[r], sem_g).wait()
            return carry

        lax.fori_loop(0, CP, g_start, 0)
        lax.fori_loop(0, CP, g_wait, 0)

        def piece_body(r, zc):
            rowi = lane * 0 + r
            for c in range(GPR):
                si = src_v[base + r, pl.ds(c * 16, 16)]
                u = plsc.load_gather(xs_v, [si])
                ei = lane + c * 16
                znew = []
                for h in range(H):
                    hi = lane * 0 + h
                    t = (plsc.load_gather(q1g, [rowi, ei, hi])
                         + plsc.load_gather(q2g, [rowi, ei, hi]))
                    t = jnp.maximum(t, 0.2 * t)
                    p = jnp.exp(t - mh[h])
                    znew.append(zc[h] + p)
                    plsc.store_scatter(stage_v, [rowi, ei, hi], p * u)
                zc = tuple(znew)
            return zc

        zacc = lax.fori_loop(0, CP, piece_body, zacc)

        def s_start(r, carry):
            pltpu.make_async_copy(stage_v.at[r],
                                  s_sh.at[dst_v.at[base + r]],
                                  sem_s).start(add=True)
            return carry

        def s_wait(r, carry):
            pltpu.make_async_copy(stage_v.at[r],
                                  s_sh.at[dst_v.at[base + r]],
                                  sem_s).wait()
            return carry

        lax.fori_loop(0, CP, s_start, 0)
        lax.fori_loop(0, CP, s_wait, 0)

    plsc.subcore_barrier()

    @pl.when(sid < 10)
    def _readback():
        pltpu.sync_copy(s_sh.at[pl.ds(sid * 1000, 1000)],
                        sp_hbm.at[cid, pl.ds(sid * 1000, 1000)])
    for h in range(H):
        zv[0, pl.ds(h * 16, 16)] = zacc[h]
    pltpu.sync_copy(zv, zt_hbm.at[wid])


def _fin_body(sp_ref, zt_ref, ws_ref, b_ref, o_ref):
    s = sp_ref[0] + sp_ref[1]
    z = jnp.sum(zt_ref[...], axis=(0, 2))
    r = s * (1.0 / z)[None, :]
    head = lax.broadcasted_iota(jnp.int32, (H, H * HD), 0)
    colh = lax.broadcasted_iota(jnp.int32, (H, H * HD), 1) // HD
    expand = jnp.where(head == colh, 1.0, 0.0)
    y = jnp.dot(r, expand, preferred_element_type=jnp.float32)
    y = y * ws_ref[...] + b_ref[...]
    o_ref[...] = jnp.where(y > 0.0, y, jnp.exp(jnp.minimum(y, 0.0)) - 1.0)


_edge_kernel = functools.partial(
    pl.kernel,
    out_type=[
        jax.ShapeDtypeStruct((NC, N, H), jnp.float32),
        jax.ShapeDtypeStruct((NW, 1, H * 16), jnp.float32),
    ],
    mesh=plsc.VectorSubcoreMesh(core_axis_name="c", subcore_axis_name="s"),
    compiler_params=pltpu.CompilerParams(needs_layout_passes=False,
                                         use_tc_tiling_on_sc=False),
    scratch_types=[
        pltpu.VMEM((N,), jnp.float32),
        pltpu.VMEM((ROWS, PW), jnp.int32),
        pltpu.VMEM((ROWS, PW), jnp.int32),
        pltpu.VMEM((CP, PW, H), jnp.float32),
        pltpu.VMEM((CP, PW, H), jnp.float32),
        pltpu.VMEM((CP, PW, H), jnp.float32),
        pltpu.VMEM((H, 16), jnp.float32),
        pltpu.VMEM((1, H * 16), jnp.float32),
        pltpu.VMEM_SHARED((N, H), jnp.float32),
        pltpu.SemaphoreType.DMA,
        pltpu.SemaphoreType.DMA,
    ],
)(_edge_body)


@jax.jit
def kernel(x, edge_index, W, a, bias):
    src = edge_index[0].astype(jnp.int32).reshape(NW, ROWS, PW)
    dst = edge_index[1].astype(jnp.int32).reshape(NW, ROWS, PW)
    wt = W.transpose(1, 0, 2).reshape(IN_F_, H * HD)
    a1f = a[:, :HD, 0].reshape(1, H * HD)
    a2f = a[:, HD:, 0].reshape(1, H * HD)

    xs2, q1, q2, mv, ws = pl.pallas_call(
        _prep_body,
        out_shape=[
            jax.ShapeDtypeStruct((N, 1), jnp.float32),
            jax.ShapeDtypeStruct((N, H), jnp.float32),
            jax.ShapeDtypeStruct((N, H), jnp.float32),
            jax.ShapeDtypeStruct((1, H), jnp.float32),
            jax.ShapeDtypeStruct((1, H * HD), jnp.float32),
        ],
    )(x, wt, a1f, a2f)

    xs = xs2.reshape(N)
    mvb = jnp.broadcast_to(mv.reshape(H, 1), (H, 16))
    zero = jnp.zeros((N, H), jnp.float32)

    sp, zt = _edge_kernel(xs, src, dst, mvb, q1, q2, zero)

    out = pl.pallas_call(
        _fin_body,
        out_shape=jax.ShapeDtypeStruct((N, H * HD), jnp.float32),
    )(sp, zt.reshape(NW, H, 16), ws, bias.reshape(1, H * HD))
    return out


# speedup vs baseline: 187.0269x; 1.0641x over previous
import functools

import jax
import jax.numpy as jnp
from jax import lax
from jax.experimental import pallas as pl
from jax.experimental.pallas import tpu as pltpu
from jax.experimental.pallas import tpu_sc as plsc

N = 10000
E = 320000
H = 8
HD = 16
IN_F_ = 128
NC = 2
NS = 16
NW = NC * NS
EPT = E // NW
PW = 80
ROWS = EPT // PW
GPR = PW // 16
CP = 25
NCHUNK = ROWS // CP


def _prep_body(x_ref, wt_ref, a1_ref, a2_ref, xs_ref, q1_ref, q2_ref,
               mv_ref, ws_ref):
    x = x_ref[...]
    xs = jnp.sum(x, axis=1, keepdims=True)
    xs_ref[...] = xs
    wsf = jnp.sum(wt_ref[...], axis=0, keepdims=True)
    ws_ref[...] = wsf

    def rnd(z):
        return z.astype(jnp.bfloat16).astype(jnp.float32)

    p2 = rnd(xs * wsf)
    grp = lax.broadcasted_iota(jnp.int32, (H * HD, H), 0) // HD
    hh = lax.broadcasted_iota(jnp.int32, (H * HD, H), 1)
    g = jnp.where(grp == hh, 1.0, 0.0)
    hp = jax.lax.Precision.HIGHEST
    q1 = jnp.dot(p2 * rnd(a1_ref[...]), g,
                 preferred_element_type=jnp.float32, precision=hp)
    q2 = jnp.dot(p2 * rnd(a2_ref[...]), g,
                 preferred_element_type=jnp.float32, precision=hp)
    q1_ref[...] = q1
    q2_ref[...] = q2
    t = (jnp.max(q1, axis=0, keepdims=True)
         + jnp.max(q2, axis=0, keepdims=True))
    mv_ref[...] = jnp.maximum(t, 0.2 * t)


def _edge_body(xs_hbm, src_hbm, dst_hbm, mb_hbm, q1_hbm, q2_hbm, zero_hbm,
               sp_hbm, zt_hbm,
               xs_v, src_v, dst_v, q1g, q2g, stage_v, mb_v, zv, s_sh,
               sem_g, sem_s):
    cid = lax.axis_index("c")
    sid = lax.axis_index("s")
    wid = sid * NC + cid

    pltpu.sync_copy(xs_hbm, xs_v)
    pltpu.sync_copy(src_hbm.at[wid], src_v)
    pltpu.sync_copy(dst_hbm.at[wid], dst_v)
    pltpu.sync_copy(mb_hbm, mb_v)
    @pl.when(sid < 10)
    def _zero():
        pltpu.sync_copy(zero_hbm.at[pl.ds(sid * 1000, 1000)],
                        s_sh.at[pl.ds(sid * 1000, 1000)])
    plsc.subcore_barrier()

    lane = lax.broadcasted_iota(jnp.int32, (16,), 0)
    mh = [mb_v[h] for h in range(H)]
    zacc = (lane * 0.0,) * H

    def g_fire(chunk, buf):
        base = chunk * CP

        def g_start(r, carry):
            pltpu.make_async_copy(q1_hbm.at[src_v.at[base + r]],
                                  q1g.at[buf, r], sem_g).start()
            pltpu.make_async_copy(q2_hbm.at[dst_v.at[base + r]],
                                  q2g.at[buf, r], sem_g).start()
            return carry

        lax.fori_loop(0, CP, g_start, 0)

    def g_drain(chunk, buf):
        base = chunk * CP

        def g_wait(r, carry):
            pltpu.make_async_copy(q1_hbm.at[src_v.at[base + r]],
                                  q1g.at[buf, r], sem_g).wait()
            pltpu.make_async_copy(q2_hbm.at[dst_v.at[base + r]],
                                  q2g.at[buf, r], sem_g).wait()
            return carry

        lax.fori_loop(0, CP, g_wait, 0)

    g_fire(0, 0)
    for chunk in range(NCHUNK):
        base = chunk * CP
        buf = chunk % 2
        if chunk + 1 < NCHUNK:
            g_fire(chunk + 1, 1 - buf)
        g_drain(chunk, buf)
        if chunk > 0:
            pbase = (chunk - 1) * CP

            def s_wait_prev(r, carry):
                pltpu.make_async_copy(stage_v.at[r],
                                      s_sh.at[dst_v.at[pbase + r]],
                                      sem_s).wait()
                return carry

            lax.fori_loop(0, CP, s_wait_prev, 0)

        def piece_body(r, zc):
            rowi = lane * 0 + r
            for c in range(GPR):
                si = src_v[base + r, pl.ds(c * 16, 16)]
                u = plsc.load_gather(xs_v, [si])
                ei = lane + c * 16
                znew = []
                for h in range(H):
                    hi = lane * 0 + h
                    t = (plsc.load_gather(q1g, [lane * 0 + buf, rowi, ei, hi])
                         + plsc.load_gather(q2g, [lane * 0 + buf, rowi, ei, hi]))
                    t = jnp.maximum(t, 0.2 * t)
                    p = jnp.exp(t - mh[h])
                    znew.append(zc[h] + p)
                    plsc.store_scatter(stage_v, [rowi, ei, hi], p * u)
                zc = tuple(znew)
            return zc

        zacc = lax.fori_loop(0, CP, piece_body, zacc)

        def s_start(r, carry):
            pltpu.make_async_copy(stage_v.at[r],
                                  s_sh.at[dst_v.at[base + r]],
                                  sem_s).start(add=True)
            return carry

        lax.fori_loop(0, CP, s_start, 0)

    fbase = (NCHUNK - 1) * CP

    def s_wait_final(r, carry):
        pltpu.make_async_copy(stage_v.at[r],
                              s_sh.at[dst_v.at[fbase + r]],
                              sem_s).wait()
        return carry

    lax.fori_loop(0, CP, s_wait_final, 0)

    plsc.subcore_barrier()

    @pl.when(sid < 10)
    def _readback():
        pltpu.sync_copy(s_sh.at[pl.ds(sid * 1000, 1000)],
                        sp_hbm.at[cid, pl.ds(sid * 1000, 1000)])
    for h in range(H):
        zv[0, pl.ds(h * 16, 16)] = zacc[h]
    pltpu.sync_copy(zv, zt_hbm.at[wid])


def _fin_body(sp_ref, zt_ref, ws_ref, b_ref, o_ref):
    s = sp_ref[0] + sp_ref[1]
    z = jnp.sum(zt_ref[...], axis=(0, 2))
    r = s * (1.0 / z)[None, :]
    head = lax.broadcasted_iota(jnp.int32, (H, H * HD), 0)
    colh = lax.broadcasted_iota(jnp.int32, (H, H * HD), 1) // HD
    expand = jnp.where(head == colh, 1.0, 0.0)
    y = jnp.dot(r, expand, preferred_element_type=jnp.float32)
    y = y * ws_ref[...] + b_ref[...]
    o_ref[...] = jnp.where(y > 0.0, y, jnp.exp(jnp.minimum(y, 0.0)) - 1.0)


_edge_kernel = functools.partial(
    pl.kernel,
    out_type=[
        jax.ShapeDtypeStruct((NC, N, H), jnp.float32),
        jax.ShapeDtypeStruct((NW, 1, H * 16), jnp.float32),
    ],
    mesh=plsc.VectorSubcoreMesh(core_axis_name="c", subcore_axis_name="s"),
    compiler_params=pltpu.CompilerParams(needs_layout_passes=False,
                                         use_tc_tiling_on_sc=False),
    scratch_types=[
        pltpu.VMEM((N,), jnp.float32),
        pltpu.VMEM((ROWS, PW), jnp.int32),
        pltpu.VMEM((ROWS, PW), jnp.int32),
        pltpu.VMEM((2, CP, PW, H), jnp.float32),
        pltpu.VMEM((2, CP, PW, H), jnp.float32),
        pltpu.VMEM((CP, PW, H), jnp.float32),
        pltpu.VMEM((H, 16), jnp.float32),
        pltpu.VMEM((1, H * 16), jnp.float32),
        pltpu.VMEM_SHARED((N, H), jnp.float32),
        pltpu.SemaphoreType.DMA,
        pltpu.SemaphoreType.DMA,
    ],
)(_edge_body)


@jax.jit
def kernel(x, edge_index, W, a, bias):
    src = edge_index[0].astype(jnp.int32).reshape(NW, ROWS, PW)
    dst = edge_index[1].astype(jnp.int32).reshape(NW, ROWS, PW)
    wt = W.transpose(1, 0, 2).reshape(IN_F_, H * HD)
    a1f = a[:, :HD, 0].reshape(1, H * HD)
    a2f = a[:, HD:, 0].reshape(1, H * HD)

    xs2, q1, q2, mv, ws = pl.pallas_call(
        _prep_body,
        out_shape=[
            jax.ShapeDtypeStruct((N, 1), jnp.float32),
            jax.ShapeDtypeStruct((N, H), jnp.float32),
            jax.ShapeDtypeStruct((N, H), jnp.float32),
            jax.ShapeDtypeStruct((1, H), jnp.float32),
            jax.ShapeDtypeStruct((1, H * HD), jnp.float32),
        ],
    )(x, wt, a1f, a2f)

    xs = xs2.reshape(N)
    mvb = jnp.broadcast_to(mv.reshape(H, 1), (H, 16))
    zero = jnp.zeros((N, H), jnp.float32)

    sp, zt = _edge_kernel(xs, src, dst, mvb, q1, q2, zero)

    out = pl.pallas_call(
        _fin_body,
        out_shape=jax.ShapeDtypeStruct((N, H * HD), jnp.float32),
    )(sp, zt.reshape(NW, H, 16), ws, bias.reshape(1, H * HD))
    return out
